# baseline (speedup 1.0000x reference)
import jax
import jax.numpy as jnp
from jax import lax
from jax.experimental import pallas as pl
from jax.experimental.pallas import tpu as pltpu
from jax.experimental.pallas import tpu_sc as plsc

_PC0, _PC1, _PC5 = -51.2, -51.2, 3.0
_GRID = 0.2
_POSW, _NEGW = 5.0, 0.1
_Y, _X = 512, 512
_B, _N = 4, 24
_BSC = 2
_BTC = _B - _BSC
_RS = 32
_NS = _Y // _RS
_RB = 64
_NR = _Y // _RB
_SL = 8


def _box_params_t(boxes):
    cxg = (boxes[..., 0] - _PC0) / _GRID
    cyg = (boxes[..., 1] - _PC1) / _GRID
    hw = (boxes[..., 3] / _GRID) / 2.0
    hl = (boxes[..., 4] / _GRID) / 2.0
    theta = boxes[..., 6]
    cos_t = jnp.cos(-theta)
    sin_t = jnp.sin(-theta)
    hv = boxes[..., 5] / (_PC5 + 2.0)
    ey = jnp.abs(sin_t) * hw + jnp.abs(cos_t) * hl
    ex = jnp.abs(cos_t) * hw + jnp.abs(sin_t) * hl
    p = jnp.stack([cxg, cyg, cos_t, sin_t, hw, hl, hv,
                   cyg - ey, cyg + ey, cxg - ex, cxg + ex], axis=1)
    return jnp.pad(p, ((0, 0), (0, 5), (0, 32 - _N)))


def _raster_body(params_hbm, gt_hbm, params_v, gtbuf):
    c = lax.axis_index("c")
    s = lax.axis_index("s")
    wid = s * 2 + c
    b = wid // _NS
    slab = wid % _NS
    row0 = slab * _RS

    pltpu.sync_copy(params_hbm.at[b], params_v)

    pvec = [[params_v[k, pl.ds(cc * 16, 16)] for k in range(11)]
            for cc in range(2)]

    zero16 = jnp.zeros((16,), jnp.float32)

    def _zero(y, _):
        for j in range(_X // 16):
            gtbuf[y, pl.ds(j * 16, 16)] = zero16
        return 0

    lax.fori_loop(0, _RS, _zero, 0)

    lane = lax.iota(jnp.int32, 16).astype(jnp.float32)

    for i in range(_N):
        cv = pvec[i // 16]
        j = i % 16
        cxg = cv[0][j]
        cyg = cv[1][j]
        cos_t = cv[2][j]
        sin_t = cv[3][j]
        hw = cv[4][j]
        hl = cv[5][j]
        hv = cv[6][j]
        ymin = cv[7][j]
        ymax = cv[8][j]
        xmin = cv[9][j]

        y0 = jnp.maximum(ymin.astype(jnp.int32), row0)
        y1 = jnp.minimum(ymax.astype(jnp.int32), row0 + (_RS - 1))
        x0 = jnp.maximum(xmin.astype(jnp.int32), 0)
        cbase = jnp.minimum((x0 >> 4) << 4, _X - 48)
        hv_v = jnp.full((16,), hv, jnp.float32)

        def _row(y, _, cxg=cxg, cyg=cyg, cos_t=cos_t, sin_t=sin_t,
                 hw=hw, hl=hl, hv_v=hv_v, cbase=cbase):
            dy = y.astype(jnp.float32) - cyg
            ys = dy * sin_t
            yc = dy * cos_t
            yl = y - row0
            for cc in range(3):
                cstart = pl.multiple_of(cbase + cc * 16, 16)
                dx = (lane + cstart.astype(jnp.float32)) - cxg
                l0 = dx * cos_t - ys
                l1 = dx * sin_t + yc
                inside = (jnp.abs(l0) <= hw) & (jnp.abs(l1) <= hl)
                old = gtbuf[yl, pl.ds(cstart, 16)]
                gtbuf[yl, pl.ds(cstart, 16)] = jnp.where(inside, hv_v, old)
            return 0

        lax.fori_loop(y0, y1 + 1, _row, 0)

    pltpu.sync_copy(gtbuf, gt_hbm.at[b, pl.ds(row0, _RS)])


def _rasterize(params_t):
    mesh = plsc.VectorSubcoreMesh(
        core_axis_name="c", subcore_axis_name="s", num_cores=2, num_subcores=16
    )
    return pl.kernel(
        _raster_body,
        out_type=jax.ShapeDtypeStruct((_BSC, _Y, _X), jnp.float32),
        mesh=mesh,
        scratch_types=[
            pltpu.VMEM((16, 32), jnp.float32),
            pltpu.VMEM((_RS, _X), jnp.float32),
        ],
    )(params_t[:_BSC])


def _loss_terms(x, gt, hm):
    e = jnp.exp(-jnp.abs(x))
    bce0 = jnp.maximum(x, 0.0) + jnp.log1p(e)
    rp = 1.0 / (1.0 + e)
    p = jnp.where(x >= 0.0, rp, 1.0 - rp)

    pos = gt > 0.0
    point = hm > 0.0
    wb = jnp.where(pos, _POSW, jnp.where(point, _NEGW, 0.0))
    vf = jnp.where(pos | point, 1.0, 0.0)

    bce = bce0 - x * gt
    omp = p + gt * (1.0 - 2.0 * p)
    focal = omp * omp * (0.75 - 0.5 * gt)
    t1 = bce * wb
    return t1, t1 * focal, vf


def _fused_body(params_ref, x_ref, hm_ref, sums_ref, gt_ref, acc_ref, vacc_ref):
    b2 = pl.program_id(0)
    r = pl.program_id(1)
    b = b2 + _BTC
    row0 = (r * _RB).astype(jnp.float32)

    gt_ref[...] = jnp.zeros((_RB, _X), jnp.float32)
    _W = 256
    _H = 40
    rowf0 = jax.lax.broadcasted_iota(jnp.int32, (_H, _W), 0).astype(jnp.float32)
    colf = jax.lax.broadcasted_iota(jnp.int32, (_H, _W), 1).astype(jnp.float32)
    row0i = r * _RB

    for i in range(_N):
        cxg = params_ref[b, 0, i]
        cyg = params_ref[b, 1, i]
        cos_t = params_ref[b, 2, i]
        sin_t = params_ref[b, 3, i]
        hw = params_ref[b, 4, i]
        hl = params_ref[b, 5, i]
        hv = params_ref[b, 6, i]
        ymin = params_ref[b, 7, i]
        ymax = params_ref[b, 8, i]
        xmin = params_ref[b, 9, i]

        x0 = jnp.maximum(xmin.astype(jnp.int32), 0)
        cwin = pl.multiple_of(jnp.minimum((x0 >> 7) << 7, _X - _W), 128)
        rel = jnp.maximum(ymin.astype(jnp.int32) - row0i, 0)
        rwin = pl.multiple_of(jnp.minimum((rel >> 3) << 3, _RB - _H), 8)

        @pl.when(jnp.logical_and(ymax >= row0, ymin <= row0 + (_RB - 1)))
        def _():
            dx = (colf + cwin.astype(jnp.float32)) - cxg
            dy = (rowf0 + (row0i + rwin).astype(jnp.float32)) - cyg
            l0 = dx * cos_t - dy * sin_t
            l1 = dx * sin_t + dy * cos_t
            inside = (jnp.abs(l0) <= hw) & (jnp.abs(l1) <= hl)
            gtw = gt_ref[pl.ds(rwin, _H), pl.ds(cwin, _W)]
            gt_ref[pl.ds(rwin, _H), pl.ds(cwin, _W)] = jnp.where(inside, hv, gtw)

    a_bce = jnp.zeros((_SL, _X), jnp.float32)
    a_foc = jnp.zeros((_SL, _X), jnp.float32)
    a_cnt = jnp.zeros((_SL, _X), jnp.float32)
    for k in range(_RB // _SL):
        sl = pl.ds(k * _SL, _SL)
        t1, t2, vf = _loss_terms(x_ref[sl, :], gt_ref[sl, :], hm_ref[sl, :])
        a_bce = a_bce + t1
        a_foc = a_foc + t2
        a_cnt = a_cnt + vf

    @pl.when(r == 0)
    def _():
        vacc_ref[0:_SL] = a_bce
        vacc_ref[_SL:2 * _SL] = a_foc
        vacc_ref[2 * _SL:3 * _SL] = a_cnt

    @pl.when(r != 0)
    def _():
        vacc_ref[0:_SL] += a_bce
        vacc_ref[_SL:2 * _SL] += a_foc
        vacc_ref[2 * _SL:3 * _SL] += a_cnt

    @pl.when(r == _NR - 1)
    def _():
        sums_ref[b2, 0] = jnp.sum(vacc_ref[0:_SL])
        sums_ref[b2, 1] = jnp.sum(vacc_ref[_SL:2 * _SL])
        sums_ref[b2, 2] = jnp.sum(vacc_ref[2 * _SL:3 * _SL])
    del acc_ref


def _fused23(params_t, x2, hm2):
    return pl.pallas_call(
        _fused_body,
        grid=(_BTC, _NR),
        in_specs=[
            pl.BlockSpec(memory_space=pltpu.SMEM),
            pl.BlockSpec((_RB, _X), lambda b2, r: ((b2 + _BTC) * _NR + r, 0)),
            pl.BlockSpec((_RB, _X), lambda b2, r: ((b2 + _BTC) * _NR + r, 0)),
        ],
        out_specs=pl.BlockSpec(memory_space=pltpu.SMEM),
        out_shape=jax.ShapeDtypeStruct((_BTC, 3), jnp.float32),
        scratch_shapes=[
            pltpu.VMEM((_RB, _X), jnp.float32),
            pltpu.SMEM((_BTC, 3), jnp.float32),
            pltpu.VMEM((3 * _SL, _X), jnp.float32),
        ],
    )(params_t, x2, hm2)


def _lossB_body(x_ref, gt_ref, hm_ref, sums23_ref, out_ref, acc_ref, vacc_ref):
    b = pl.program_id(0)

    a_bce = jnp.zeros((_SL, _X), jnp.float32)
    a_foc = jnp.zeros((_SL, _X), jnp.float32)
    a_cnt = jnp.zeros((_SL, _X), jnp.float32)
    for k in range(_Y // _SL):
        sl = pl.ds(k * _SL, _SL)
        t1, t2, vf = _loss_terms(x_ref[sl, :], gt_ref[sl, :], hm_ref[sl, :])
        a_bce = a_bce + t1
        a_foc = a_foc + t2
        a_cnt = a_cnt + vf

    vacc_ref[0:_SL] = a_bce
    vacc_ref[_SL:2 * _SL] = a_foc
    vacc_ref[2 * _SL:3 * _SL] = a_cnt
    acc_ref[b, 0] = jnp.sum(vacc_ref[0:_SL])
    acc_ref[b, 1] = jnp.sum(vacc_ref[_SL:2 * _SL])
    acc_ref[b, 2] = jnp.sum(vacc_ref[2 * _SL:3 * _SL])

    @pl.when(b == _BSC - 1)
    def _():
        total = jnp.float32(0.0)
        vs = jnp.float32(0.0)
        for bb in range(_B):
            if bb < _BSC:
                sb = acc_ref[bb, 0]
                sf = acc_ref[bb, 1]
                cnt = acc_ref[bb, 2]
            else:
                sb = sums23_ref[bb - _BSC, 0]
                sf = sums23_ref[bb - _BSC, 1]
                cnt = sums23_ref[bb - _BSC, 2]
            denom = jnp.maximum(cnt, 1.0)
            comb = 0.5 * (sb + sf) / denom
            has_valid = (cnt > 0.0).astype(jnp.float32)
            total = total + comb * has_valid
            vs = vs + has_valid
        out_ref[0, 0] = jnp.where(vs > 0.0, total / jnp.maximum(vs, 1.0), total)


def _lossB(x2, gt01, hm2, sums23):
    return pl.pallas_call(
        _lossB_body,
        grid=(_BSC,),
        in_specs=[
            pl.BlockSpec((_Y, _X), lambda b: (b, 0)),
            pl.BlockSpec((_Y, _X), lambda b: (b, 0)),
            pl.BlockSpec((_Y, _X), lambda b: (b, 0)),
            pl.BlockSpec(memory_space=pltpu.SMEM),
        ],
        out_specs=pl.BlockSpec(memory_space=pltpu.SMEM),
        out_shape=jax.ShapeDtypeStruct((1, 1), jnp.float32),
        scratch_shapes=[
            pltpu.SMEM((_BSC, 3), jnp.float32),
            pltpu.VMEM((3 * _SL, _X), jnp.float32),
        ],
    )(x2, gt01, hm2, sums23)


def kernel(attention_logits, gt_bboxes_3d, height_maps):
    params_t = _box_params_t(gt_bboxes_3d)
    x2 = attention_logits.reshape(_B * _Y, _X)
    hm2 = height_maps.reshape(_B * _Y, _X)
    gt01 = _rasterize(params_t).reshape(_BSC * _Y, _X)
    sums23 = _fused23(params_t, x2, hm2)
    return _lossB(x2, gt01, hm2, sums23)[0, 0]


# speedup vs baseline: 1.1869x; 1.1869x over previous
import jax
import jax.numpy as jnp
from jax import lax
from jax.experimental import pallas as pl
from jax.experimental.pallas import tpu as pltpu
from jax.experimental.pallas import tpu_sc as plsc

_PC0, _PC1, _PC5 = -51.2, -51.2, 3.0
_GRID = 0.2
_POSW, _NEGW = 5.0, 0.1
_Y, _X = 512, 512
_B, _N = 4, 24
_BSC = 2
_BTC = _B - _BSC
_RS = 32
_NS = _Y // _RS
_RB = 64
_NR = _Y // _RB
_SL = 8


def _box_params_t(boxes):
    cxg = (boxes[..., 0] - _PC0) / _GRID
    cyg = (boxes[..., 1] - _PC1) / _GRID
    hw = (boxes[..., 3] / _GRID) / 2.0
    hl = (boxes[..., 4] / _GRID) / 2.0
    theta = boxes[..., 6]
    cos_t = jnp.cos(-theta)
    sin_t = jnp.sin(-theta)
    hv = boxes[..., 5] / (_PC5 + 2.0)
    ey = jnp.abs(sin_t) * hw + jnp.abs(cos_t) * hl
    ex = jnp.abs(cos_t) * hw + jnp.abs(sin_t) * hl
    p = jnp.stack([cxg, cyg, cos_t, sin_t, hw, hl, hv,
                   cyg - ey, cyg + ey, cxg - ex, cxg + ex], axis=1)
    return jnp.pad(p, ((0, 0), (0, 5), (0, 32 - _N)))


def _raster_body(params_hbm, gt_hbm, params_v, gtbuf):
    c = lax.axis_index("c")
    s = lax.axis_index("s")
    wid = s * 2 + c
    b = wid // _NS
    slab = wid % _NS
    row0 = slab * _RS

    pltpu.sync_copy(params_hbm.at[b], params_v)

    pvec = [[params_v[k, pl.ds(cc * 16, 16)] for k in range(11)]
            for cc in range(2)]

    zero16 = jnp.zeros((16,), jnp.float32)

    def _zero(y, _):
        for j in range(_X // 16):
            gtbuf[y, pl.ds(j * 16, 16)] = zero16
        return 0

    lax.fori_loop(0, _RS, _zero, 0)

    lane = lax.iota(jnp.int32, 16).astype(jnp.float32)

    for i in range(_N):
        cv = pvec[i // 16]
        j = i % 16
        cxg = cv[0][j]
        cyg = cv[1][j]
        cos_t = cv[2][j]
        sin_t = cv[3][j]
        hw = cv[4][j]
        hl = cv[5][j]
        hv = cv[6][j]
        ymin = cv[7][j]
        ymax = cv[8][j]
        xmin = cv[9][j]

        y0 = jnp.maximum(ymin.astype(jnp.int32), row0)
        y1 = jnp.minimum(ymax.astype(jnp.int32), row0 + (_RS - 1))
        x0 = jnp.maximum(xmin.astype(jnp.int32), 0)
        cbase = jnp.minimum((x0 >> 4) << 4, _X - 48)
        hv_v = jnp.full((16,), hv, jnp.float32)

        def _row(y, _, cxg=cxg, cyg=cyg, cos_t=cos_t, sin_t=sin_t,
                 hw=hw, hl=hl, hv_v=hv_v, cbase=cbase):
            dy = y.astype(jnp.float32) - cyg
            ys = dy * sin_t
            yc = dy * cos_t
            yl = y - row0
            for cc in range(3):
                cstart = pl.multiple_of(cbase + cc * 16, 16)
                dx = (lane + cstart.astype(jnp.float32)) - cxg
                l0 = dx * cos_t - ys
                l1 = dx * sin_t + yc
                inside = (jnp.abs(l0) <= hw) & (jnp.abs(l1) <= hl)
                old = gtbuf[yl, pl.ds(cstart, 16)]
                gtbuf[yl, pl.ds(cstart, 16)] = jnp.where(inside, hv_v, old)
            return 0

        lax.fori_loop(y0, y1 + 1, _row, 0)

    pltpu.sync_copy(gtbuf, gt_hbm.at[b, pl.ds(row0, _RS)])


def _rasterize(params_t):
    mesh = plsc.VectorSubcoreMesh(
        core_axis_name="c", subcore_axis_name="s", num_cores=2, num_subcores=16
    )
    return pl.kernel(
        _raster_body,
        out_type=jax.ShapeDtypeStruct((_BSC, _Y, _X), jnp.float32),
        mesh=mesh,
        scratch_types=[
            pltpu.VMEM((16, 32), jnp.float32),
            pltpu.VMEM((_RS, _X), jnp.float32),
        ],
    )(params_t[:_BSC])


def _loss_terms(x, gt, hm):
    e = jnp.exp(-jnp.abs(x))
    bce0 = jnp.maximum(x, 0.0) + jnp.log1p(e)
    rp = 1.0 / (1.0 + e)
    p = jnp.where(x >= 0.0, rp, 1.0 - rp)

    pos = gt > 0.0
    point = hm > 0.0
    wb = jnp.where(pos, _POSW, jnp.where(point, _NEGW, 0.0))
    vf = jnp.where(pos | point, 1.0, 0.0)

    bce = bce0 - x * gt
    omp = p + gt * (1.0 - 2.0 * p)
    focal = omp * omp * (0.75 - 0.5 * gt)
    t1 = bce * wb
    return t1, t1 * focal, vf


def _fused_body(params_ref, x_ref, hm_ref, sums_ref, gt_ref):
    b2 = pl.program_id(0)
    b = b2 + _BTC

    gt_ref[...] = jnp.zeros((_Y, _X), jnp.float32)
    _W = 256
    _H = 40
    rowf0 = jax.lax.broadcasted_iota(jnp.int32, (_H, _W), 0).astype(jnp.float32)
    colf = jax.lax.broadcasted_iota(jnp.int32, (_H, _W), 1).astype(jnp.float32)

    for i in range(_N):
        cxg = params_ref[b, 0, i]
        cyg = params_ref[b, 1, i]
        cos_t = params_ref[b, 2, i]
        sin_t = params_ref[b, 3, i]
        hw = params_ref[b, 4, i]
        hl = params_ref[b, 5, i]
        hv = params_ref[b, 6, i]
        ymin = params_ref[b, 7, i]
        xmin = params_ref[b, 9, i]

        x0 = jnp.maximum(xmin.astype(jnp.int32), 0)
        cwin = pl.multiple_of(jnp.minimum((x0 >> 7) << 7, _X - _W), 128)
        y0 = jnp.maximum(ymin.astype(jnp.int32), 0)
        rwin = pl.multiple_of(jnp.minimum((y0 >> 3) << 3, _Y - _H), 8)

        dx = (colf + cwin.astype(jnp.float32)) - cxg
        dy = (rowf0 + rwin.astype(jnp.float32)) - cyg
        l0 = dx * cos_t - dy * sin_t
        l1 = dx * sin_t + dy * cos_t
        inside = (jnp.abs(l0) <= hw) & (jnp.abs(l1) <= hl)
        gtw = gt_ref[pl.ds(rwin, _H), pl.ds(cwin, _W)]
        gt_ref[pl.ds(rwin, _H), pl.ds(cwin, _W)] = jnp.where(inside, hv, gtw)

    a_bce = jnp.zeros((_SL, _X), jnp.float32)
    a_foc = jnp.zeros((_SL, _X), jnp.float32)
    a_cnt = jnp.zeros((_SL, _X), jnp.float32)
    for k in range(_Y // _SL):
        sl = pl.ds(k * _SL, _SL)
        t1, t2, vf = _loss_terms(x_ref[sl, :], gt_ref[sl, :], hm_ref[sl, :])
        a_bce = a_bce + t1
        a_foc = a_foc + t2
        a_cnt = a_cnt + vf

    sums_ref[b2, 0] = jnp.sum(a_bce)
    sums_ref[b2, 1] = jnp.sum(a_foc)
    sums_ref[b2, 2] = jnp.sum(a_cnt)


def _fused23(params_t, x2, hm2):
    return pl.pallas_call(
        _fused_body,
        grid=(_BTC,),
        in_specs=[
            pl.BlockSpec(memory_space=pltpu.SMEM),
            pl.BlockSpec((_Y, _X), lambda b2: (b2 + _BTC, 0)),
            pl.BlockSpec((_Y, _X), lambda b2: (b2 + _BTC, 0)),
        ],
        out_specs=pl.BlockSpec(memory_space=pltpu.SMEM),
        out_shape=jax.ShapeDtypeStruct((_BTC, 3), jnp.float32),
        scratch_shapes=[
            pltpu.VMEM((_Y, _X), jnp.float32),
        ],
    )(params_t, x2, hm2)


def _lossB_body(x_ref, gt_ref, hm_ref, sums23_ref, out_ref, acc_ref, vacc_ref):
    b = pl.program_id(0)

    a_bce = jnp.zeros((_SL, _X), jnp.float32)
    a_foc = jnp.zeros((_SL, _X), jnp.float32)
    a_cnt = jnp.zeros((_SL, _X), jnp.float32)
    for k in range(_Y // _SL):
        sl = pl.ds(k * _SL, _SL)
        t1, t2, vf = _loss_terms(x_ref[sl, :], gt_ref[sl, :], hm_ref[sl, :])
        a_bce = a_bce + t1
        a_foc = a_foc + t2
        a_cnt = a_cnt + vf

    vacc_ref[0:_SL] = a_bce
    vacc_ref[_SL:2 * _SL] = a_foc
    vacc_ref[2 * _SL:3 * _SL] = a_cnt
    acc_ref[b, 0] = jnp.sum(vacc_ref[0:_SL])
    acc_ref[b, 1] = jnp.sum(vacc_ref[_SL:2 * _SL])
    acc_ref[b, 2] = jnp.sum(vacc_ref[2 * _SL:3 * _SL])

    @pl.when(b == _BSC - 1)
    def _():
        total = jnp.float32(0.0)
        vs = jnp.float32(0.0)
        for bb in range(_B):
            if bb < _BSC:
                sb = acc_ref[bb, 0]
                sf = acc_ref[bb, 1]
                cnt = acc_ref[bb, 2]
            else:
                sb = sums23_ref[bb - _BSC, 0]
                sf = sums23_ref[bb - _BSC, 1]
                cnt = sums23_ref[bb - _BSC, 2]
            denom = jnp.maximum(cnt, 1.0)
            comb = 0.5 * (sb + sf) / denom
            has_valid = (cnt > 0.0).astype(jnp.float32)
            total = total + comb * has_valid
            vs = vs + has_valid
        out_ref[0, 0] = jnp.where(vs > 0.0, total / jnp.maximum(vs, 1.0), total)


def _lossB(x2, gt01, hm2, sums23):
    return pl.pallas_call(
        _lossB_body,
        grid=(_BSC,),
        in_specs=[
            pl.BlockSpec((_Y, _X), lambda b: (b, 0)),
            pl.BlockSpec((_Y, _X), lambda b: (b, 0)),
            pl.BlockSpec((_Y, _X), lambda b: (b, 0)),
            pl.BlockSpec(memory_space=pltpu.SMEM),
        ],
        out_specs=pl.BlockSpec(memory_space=pltpu.SMEM),
        out_shape=jax.ShapeDtypeStruct((1, 1), jnp.float32),
        scratch_shapes=[
            pltpu.SMEM((_BSC, 3), jnp.float32),
            pltpu.VMEM((3 * _SL, _X), jnp.float32),
        ],
    )(x2, gt01, hm2, sums23)


def kernel(attention_logits, gt_bboxes_3d, height_maps):
    params_t = _box_params_t(gt_bboxes_3d)
    x2 = attention_logits.reshape(_B * _Y, _X)
    hm2 = height_maps.reshape(_B * _Y, _X)
    gt01 = _rasterize(params_t).reshape(_BSC * _Y, _X)
    sums23 = _fused23(params_t, x2, hm2)
    return _lossB(x2, gt01, hm2, sums23)[0, 0]


# speedup vs baseline: 1.1984x; 1.0098x over previous
import jax
import jax.numpy as jnp
from jax import lax
from jax.experimental import pallas as pl
from jax.experimental.pallas import tpu as pltpu
from jax.experimental.pallas import tpu_sc as plsc

_PC0, _PC1, _PC5 = -51.2, -51.2, 3.0
_GRID = 0.2
_POSW, _NEGW = 5.0, 0.1
_Y, _X = 512, 512
_B, _N = 4, 24
_BSC = 2
_BTC = _B - _BSC
_RS = 32
_NS = _Y // _RS
_RB = 64
_NR = _Y // _RB
_SL = 8


def _box_params_t(boxes):
    cxg = (boxes[..., 0] - _PC0) / _GRID
    cyg = (boxes[..., 1] - _PC1) / _GRID
    hw = (boxes[..., 3] / _GRID) / 2.0
    hl = (boxes[..., 4] / _GRID) / 2.0
    theta = boxes[..., 6]
    cos_t = jnp.cos(-theta)
    sin_t = jnp.sin(-theta)
    hv = boxes[..., 5] / (_PC5 + 2.0)
    ey = jnp.abs(sin_t) * hw + jnp.abs(cos_t) * hl
    ex = jnp.abs(cos_t) * hw + jnp.abs(sin_t) * hl
    p = jnp.stack([cxg, cyg, cos_t, sin_t, hw, hl, hv,
                   cyg - ey, cyg + ey, cxg - ex, cxg + ex], axis=1)
    return jnp.pad(p, ((0, 0), (0, 5), (0, 32 - _N)))


def _raster_body(params_hbm, gt_hbm, params_v, gtbuf):
    c = lax.axis_index("c")
    s = lax.axis_index("s")
    wid = s * 2 + c
    b = wid // _NS
    slab = wid % _NS
    row0 = slab * _RS

    pltpu.sync_copy(params_hbm.at[b], params_v)

    pvec = [[params_v[k, pl.ds(cc * 16, 16)] for k in range(11)]
            for cc in range(2)]

    zero16 = jnp.zeros((16,), jnp.float32)

    def _zero(y, _):
        for j in range(_X // 16):
            gtbuf[y, pl.ds(j * 16, 16)] = zero16
        return 0

    lax.fori_loop(0, _RS, _zero, 0)

    lane = lax.iota(jnp.int32, 16).astype(jnp.float32)

    for i in range(_N):
        cv = pvec[i // 16]
        j = i % 16
        cxg = cv[0][j]
        cyg = cv[1][j]
        cos_t = cv[2][j]
        sin_t = cv[3][j]
        hw = cv[4][j]
        hl = cv[5][j]
        hv = cv[6][j]
        ymin = cv[7][j]
        ymax = cv[8][j]
        xmin = cv[9][j]

        y0 = jnp.maximum(ymin.astype(jnp.int32), row0)
        y1 = jnp.minimum(ymax.astype(jnp.int32), row0 + (_RS - 1))
        x0 = jnp.maximum(xmin.astype(jnp.int32), 0)
        cbase = jnp.minimum((x0 >> 4) << 4, _X - 48)
        hv_v = jnp.full((16,), hv, jnp.float32)

        def _row(y, _, cxg=cxg, cyg=cyg, cos_t=cos_t, sin_t=sin_t,
                 hw=hw, hl=hl, hv_v=hv_v, cbase=cbase):
            dy = y.astype(jnp.float32) - cyg
            ys = dy * sin_t
            yc = dy * cos_t
            yl = y - row0
            for cc in range(3):
                cstart = pl.multiple_of(cbase + cc * 16, 16)
                dx = (lane + cstart.astype(jnp.float32)) - cxg
                l0 = dx * cos_t - ys
                l1 = dx * sin_t + yc
                inside = (jnp.abs(l0) <= hw) & (jnp.abs(l1) <= hl)
                old = gtbuf[yl, pl.ds(cstart, 16)]
                gtbuf[yl, pl.ds(cstart, 16)] = jnp.where(inside, hv_v, old)
            return 0

        lax.fori_loop(y0, y1 + 1, _row, 0)

    pltpu.sync_copy(gtbuf, gt_hbm.at[b, pl.ds(row0, _RS)])


def _rasterize(params_t):
    mesh = plsc.VectorSubcoreMesh(
        core_axis_name="c", subcore_axis_name="s", num_cores=2, num_subcores=16
    )
    return pl.kernel(
        _raster_body,
        out_type=jax.ShapeDtypeStruct((_BSC, _Y, _X), jnp.float32),
        mesh=mesh,
        scratch_types=[
            pltpu.VMEM((16, 32), jnp.float32),
            pltpu.VMEM((_RS, _X), jnp.float32),
        ],
    )(params_t)


def _loss_terms(x, gt, hm):
    e = jnp.exp(-jnp.abs(x))
    bce0 = jnp.maximum(x, 0.0) + jnp.log1p(e)
    rp = 1.0 / (1.0 + e)
    p = jnp.where(x >= 0.0, rp, 1.0 - rp)

    pos = gt > 0.0
    point = hm > 0.0
    wb = jnp.where(pos, _POSW, jnp.where(point, _NEGW, 0.0))
    vf = jnp.where(pos | point, 1.0, 0.0)

    bce = bce0 - x * gt
    omp = p + gt * (1.0 - 2.0 * p)
    focal = omp * omp * (0.75 - 0.5 * gt)
    t1 = bce * wb
    return t1, t1 * focal, vf


def _fused_body(params_ref, x_ref, hm_ref, sums_ref, gt_ref):
    b2 = pl.program_id(0)
    b = b2 + _BTC

    gt_ref[...] = jnp.zeros((_Y, _X), jnp.float32)
    _W = 256
    _H = 40
    rowf0 = jax.lax.broadcasted_iota(jnp.int32, (_H, _W), 0).astype(jnp.float32)
    colf = jax.lax.broadcasted_iota(jnp.int32, (_H, _W), 1).astype(jnp.float32)

    for i in range(_N):
        cxg = params_ref[b, 0, i]
        cyg = params_ref[b, 1, i]
        cos_t = params_ref[b, 2, i]
        sin_t = params_ref[b, 3, i]
        hw = params_ref[b, 4, i]
        hl = params_ref[b, 5, i]
        hv = params_ref[b, 6, i]
        ymin = params_ref[b, 7, i]
        xmin = params_ref[b, 9, i]

        x0 = jnp.maximum(xmin.astype(jnp.int32), 0)
        cwin = pl.multiple_of(jnp.minimum((x0 >> 7) << 7, _X - _W), 128)
        y0 = jnp.maximum(ymin.astype(jnp.int32), 0)
        rwin = pl.multiple_of(jnp.minimum((y0 >> 3) << 3, _Y - _H), 8)

        dx = (colf + cwin.astype(jnp.float32)) - cxg
        dy = (rowf0 + rwin.astype(jnp.float32)) - cyg
        l0 = dx * cos_t - dy * sin_t
        l1 = dx * sin_t + dy * cos_t
        inside = (jnp.abs(l0) <= hw) & (jnp.abs(l1) <= hl)
        gtw = gt_ref[pl.ds(rwin, _H), pl.ds(cwin, _W)]
        gt_ref[pl.ds(rwin, _H), pl.ds(cwin, _W)] = jnp.where(inside, hv, gtw)

    a_bce = jnp.zeros((_SL, _X), jnp.float32)
    a_foc = jnp.zeros((_SL, _X), jnp.float32)
    a_cnt = jnp.zeros((_SL, _X), jnp.float32)
    for k in range(_Y // _SL):
        sl = pl.ds(k * _SL, _SL)
        t1, t2, vf = _loss_terms(x_ref[sl, :], gt_ref[sl, :], hm_ref[sl, :])
        a_bce = a_bce + t1
        a_foc = a_foc + t2
        a_cnt = a_cnt + vf

    sums_ref[b2, 0] = jnp.sum(a_bce)
    sums_ref[b2, 1] = jnp.sum(a_foc)
    sums_ref[b2, 2] = jnp.sum(a_cnt)


def _fused23(params_t, x2, hm2):
    return pl.pallas_call(
        _fused_body,
        grid=(_BTC,),
        in_specs=[
            pl.BlockSpec(memory_space=pltpu.SMEM),
            pl.BlockSpec((_Y, _X), lambda b2: (b2 + _BTC, 0)),
            pl.BlockSpec((_Y, _X), lambda b2: (b2 + _BTC, 0)),
        ],
        out_specs=pl.BlockSpec(memory_space=pltpu.SMEM),
        out_shape=jax.ShapeDtypeStruct((_BTC, 3), jnp.float32),
        scratch_shapes=[
            pltpu.VMEM((_Y, _X), jnp.float32),
        ],
    )(params_t, x2, hm2)


def _lossB_body(x_ref, gt_ref, hm_ref, sums23_ref, out_ref, acc_ref, vacc_ref):
    b = pl.program_id(0)

    a_bce = jnp.zeros((_SL, _X), jnp.float32)
    a_foc = jnp.zeros((_SL, _X), jnp.float32)
    a_cnt = jnp.zeros((_SL, _X), jnp.float32)
    for k in range(_Y // _SL):
        sl = pl.ds(k * _SL, _SL)
        t1, t2, vf = _loss_terms(x_ref[sl, :], gt_ref[sl, :], hm_ref[sl, :])
        a_bce = a_bce + t1
        a_foc = a_foc + t2
        a_cnt = a_cnt + vf

    vacc_ref[0:_SL] = a_bce
    vacc_ref[_SL:2 * _SL] = a_foc
    vacc_ref[2 * _SL:3 * _SL] = a_cnt
    acc_ref[b, 0] = jnp.sum(vacc_ref[0:_SL])
    acc_ref[b, 1] = jnp.sum(vacc_ref[_SL:2 * _SL])
    acc_ref[b, 2] = jnp.sum(vacc_ref[2 * _SL:3 * _SL])

    @pl.when(b == _BSC - 1)
    def _():
        total = jnp.float32(0.0)
        vs = jnp.float32(0.0)
        for bb in range(_B):
            if bb < _BSC:
                sb = acc_ref[bb, 0]
                sf = acc_ref[bb, 1]
                cnt = acc_ref[bb, 2]
            else:
                sb = sums23_ref[bb - _BSC, 0]
                sf = sums23_ref[bb - _BSC, 1]
                cnt = sums23_ref[bb - _BSC, 2]
            denom = jnp.maximum(cnt, 1.0)
            comb = 0.5 * (sb + sf) / denom
            has_valid = (cnt > 0.0).astype(jnp.float32)
            total = total + comb * has_valid
            vs = vs + has_valid
        out_ref[0, 0] = jnp.where(vs > 0.0, total / jnp.maximum(vs, 1.0), total)


def _lossB(x2, gt01, hm2, sums23):
    return pl.pallas_call(
        _lossB_body,
        grid=(_BSC,),
        in_specs=[
            pl.BlockSpec((_Y, _X), lambda b: (b, 0)),
            pl.BlockSpec((_Y, _X), lambda b: (b, 0)),
            pl.BlockSpec((_Y, _X), lambda b: (b, 0)),
            pl.BlockSpec(memory_space=pltpu.SMEM),
        ],
        out_specs=pl.BlockSpec(memory_space=pltpu.SMEM),
        out_shape=jax.ShapeDtypeStruct((1, 1), jnp.float32),
        scratch_shapes=[
            pltpu.SMEM((_BSC, 3), jnp.float32),
            pltpu.VMEM((3 * _SL, _X), jnp.float32),
        ],
    )(x2, gt01, hm2, sums23)


def kernel(attention_logits, gt_bboxes_3d, height_maps):
    params_t = _box_params_t(gt_bboxes_3d)
    x2 = attention_logits.reshape(_B * _Y, _X)
    hm2 = height_maps.reshape(_B * _Y, _X)
    gt01 = _rasterize(params_t).reshape(_BSC * _Y, _X)
    sums23 = _fused23(params_t, x2, hm2)
    return _lossB(x2, gt01, hm2, sums23)[0, 0]


# speedup vs baseline: 1.2818x; 1.0695x over previous
import jax
import jax.numpy as jnp
from jax import lax
from jax.experimental import pallas as pl
from jax.experimental.pallas import tpu as pltpu
from jax.experimental.pallas import tpu_sc as plsc

_PC0, _PC1, _PC5 = -51.2, -51.2, 3.0
_GRID = 0.2
_POSW, _NEGW = 5.0, 0.1
_Y, _X = 512, 512
_B, _N = 4, 24
_BSC = 1
_BTC = _B - _BSC
_RS = _BSC * _Y // 32
_NS = _Y // _RS
_RB = 64
_NR = _Y // _RB
_SL = 8


def _box_params_t(boxes):
    cxg = (boxes[..., 0] - _PC0) / _GRID
    cyg = (boxes[..., 1] - _PC1) / _GRID
    hw = (boxes[..., 3] / _GRID) / 2.0
    hl = (boxes[..., 4] / _GRID) / 2.0
    theta = boxes[..., 6]
    cos_t = jnp.cos(-theta)
    sin_t = jnp.sin(-theta)
    hv = boxes[..., 5] / (_PC5 + 2.0)
    ey = jnp.abs(sin_t) * hw + jnp.abs(cos_t) * hl
    ex = jnp.abs(cos_t) * hw + jnp.abs(sin_t) * hl
    p = jnp.stack([cxg, cyg, cos_t, sin_t, hw, hl, hv,
                   cyg - ey, cyg + ey, cxg - ex, cxg + ex], axis=1)
    return jnp.pad(p, ((0, 0), (0, 5), (0, 32 - _N)))


def _raster_body(params_hbm, gt_hbm, params_v, gtbuf):
    c = lax.axis_index("c")
    s = lax.axis_index("s")
    wid = s * 2 + c
    b = wid // _NS
    slab = wid % _NS
    row0 = slab * _RS

    pltpu.sync_copy(params_hbm.at[b], params_v)

    pvec = [[params_v[k, pl.ds(cc * 16, 16)] for k in range(11)]
            for cc in range(2)]

    zero16 = jnp.zeros((16,), jnp.float32)

    def _zero(y, _):
        for j in range(_X // 16):
            gtbuf[y, pl.ds(j * 16, 16)] = zero16
        return 0

    lax.fori_loop(0, _RS, _zero, 0)

    lane = lax.iota(jnp.int32, 16).astype(jnp.float32)

    for i in range(_N):
        cv = pvec[i // 16]
        j = i % 16
        cxg = cv[0][j]
        cyg = cv[1][j]
        cos_t = cv[2][j]
        sin_t = cv[3][j]
        hw = cv[4][j]
        hl = cv[5][j]
        hv = cv[6][j]
        ymin = cv[7][j]
        ymax = cv[8][j]
        xmin = cv[9][j]

        y0 = jnp.maximum(ymin.astype(jnp.int32), row0)
        y1 = jnp.minimum(ymax.astype(jnp.int32), row0 + (_RS - 1))
        x0 = jnp.maximum(xmin.astype(jnp.int32), 0)
        cbase = jnp.minimum((x0 >> 4) << 4, _X - 48)
        hv_v = jnp.full((16,), hv, jnp.float32)

        def _row(y, _, cxg=cxg, cyg=cyg, cos_t=cos_t, sin_t=sin_t,
                 hw=hw, hl=hl, hv_v=hv_v, cbase=cbase):
            dy = y.astype(jnp.float32) - cyg
            ys = dy * sin_t
            yc = dy * cos_t
            yl = y - row0
            for cc in range(3):
                cstart = pl.multiple_of(cbase + cc * 16, 16)
                dx = (lane + cstart.astype(jnp.float32)) - cxg
                l0 = dx * cos_t - ys
                l1 = dx * sin_t + yc
                inside = (jnp.abs(l0) <= hw) & (jnp.abs(l1) <= hl)
                old = gtbuf[yl, pl.ds(cstart, 16)]
                gtbuf[yl, pl.ds(cstart, 16)] = jnp.where(inside, hv_v, old)
            return 0

        lax.fori_loop(y0, y1 + 1, _row, 0)

    pltpu.sync_copy(gtbuf, gt_hbm.at[b, pl.ds(row0, _RS)])


def _rasterize(params_t):
    mesh = plsc.VectorSubcoreMesh(
        core_axis_name="c", subcore_axis_name="s", num_cores=2, num_subcores=16
    )
    return pl.kernel(
        _raster_body,
        out_type=jax.ShapeDtypeStruct((_BSC, _Y, _X), jnp.float32),
        mesh=mesh,
        scratch_types=[
            pltpu.VMEM((16, 32), jnp.float32),
            pltpu.VMEM((_RS, _X), jnp.float32),
        ],
    )(params_t)


def _loss_terms(x, gt, hm):
    e = jnp.exp(-jnp.abs(x))
    bce0 = jnp.maximum(x, 0.0) + jnp.log1p(e)
    rp = 1.0 / (1.0 + e)
    p = jnp.where(x >= 0.0, rp, 1.0 - rp)

    pos = gt > 0.0
    point = hm > 0.0
    wb = jnp.where(pos, _POSW, jnp.where(point, _NEGW, 0.0))
    vf = jnp.where(pos | point, 1.0, 0.0)

    bce = bce0 - x * gt
    omp = p + gt * (1.0 - 2.0 * p)
    focal = omp * omp * (0.75 - 0.5 * gt)
    t1 = bce * wb
    return t1, t1 * focal, vf


def _fused_body(params_ref, x_ref, hm_ref, sums_ref, gt_ref):
    b2 = pl.program_id(0)
    b = b2 + _BTC

    gt_ref[...] = jnp.zeros((_Y, _X), jnp.float32)
    _W = 256
    _H = 40
    rowf0 = jax.lax.broadcasted_iota(jnp.int32, (_H, _W), 0).astype(jnp.float32)
    colf = jax.lax.broadcasted_iota(jnp.int32, (_H, _W), 1).astype(jnp.float32)

    for i in range(_N):
        cxg = params_ref[b, 0, i]
        cyg = params_ref[b, 1, i]
        cos_t = params_ref[b, 2, i]
        sin_t = params_ref[b, 3, i]
        hw = params_ref[b, 4, i]
        hl = params_ref[b, 5, i]
        hv = params_ref[b, 6, i]
        ymin = params_ref[b, 7, i]
        xmin = params_ref[b, 9, i]

        x0 = jnp.maximum(xmin.astype(jnp.int32), 0)
        cwin = pl.multiple_of(jnp.minimum((x0 >> 7) << 7, _X - _W), 128)
        y0 = jnp.maximum(ymin.astype(jnp.int32), 0)
        rwin = pl.multiple_of(jnp.minimum((y0 >> 3) << 3, _Y - _H), 8)

        dx = (colf + cwin.astype(jnp.float32)) - cxg
        dy = (rowf0 + rwin.astype(jnp.float32)) - cyg
        l0 = dx * cos_t - dy * sin_t
        l1 = dx * sin_t + dy * cos_t
        inside = (jnp.abs(l0) <= hw) & (jnp.abs(l1) <= hl)
        gtw = gt_ref[pl.ds(rwin, _H), pl.ds(cwin, _W)]
        gt_ref[pl.ds(rwin, _H), pl.ds(cwin, _W)] = jnp.where(inside, hv, gtw)

    a_bce = jnp.zeros((_SL, _X), jnp.float32)
    a_foc = jnp.zeros((_SL, _X), jnp.float32)
    a_cnt = jnp.zeros((_SL, _X), jnp.float32)
    for k in range(_Y // _SL):
        sl = pl.ds(k * _SL, _SL)
        t1, t2, vf = _loss_terms(x_ref[sl, :], gt_ref[sl, :], hm_ref[sl, :])
        a_bce = a_bce + t1
        a_foc = a_foc + t2
        a_cnt = a_cnt + vf

    sums_ref[b2, 0] = jnp.sum(a_bce)
    sums_ref[b2, 1] = jnp.sum(a_foc)
    sums_ref[b2, 2] = jnp.sum(a_cnt)


def _fused23(params_t, x2, hm2):
    return pl.pallas_call(
        _fused_body,
        grid=(_BTC,),
        in_specs=[
            pl.BlockSpec(memory_space=pltpu.SMEM),
            pl.BlockSpec((_Y, _X), lambda b2: (b2 + _BTC, 0)),
            pl.BlockSpec((_Y, _X), lambda b2: (b2 + _BTC, 0)),
        ],
        out_specs=pl.BlockSpec(memory_space=pltpu.SMEM),
        out_shape=jax.ShapeDtypeStruct((_BTC, 3), jnp.float32),
        scratch_shapes=[
            pltpu.VMEM((_Y, _X), jnp.float32),
        ],
    )(params_t, x2, hm2)


def _lossB_body(x_ref, gt_ref, hm_ref, sums23_ref, out_ref, acc_ref, vacc_ref):
    b = pl.program_id(0)

    a_bce = jnp.zeros((_SL, _X), jnp.float32)
    a_foc = jnp.zeros((_SL, _X), jnp.float32)
    a_cnt = jnp.zeros((_SL, _X), jnp.float32)
    for k in range(_Y // _SL):
        sl = pl.ds(k * _SL, _SL)
        t1, t2, vf = _loss_terms(x_ref[sl, :], gt_ref[sl, :], hm_ref[sl, :])
        a_bce = a_bce + t1
        a_foc = a_foc + t2
        a_cnt = a_cnt + vf

    vacc_ref[0:_SL] = a_bce
    vacc_ref[_SL:2 * _SL] = a_foc
    vacc_ref[2 * _SL:3 * _SL] = a_cnt
    acc_ref[b, 0] = jnp.sum(vacc_ref[0:_SL])
    acc_ref[b, 1] = jnp.sum(vacc_ref[_SL:2 * _SL])
    acc_ref[b, 2] = jnp.sum(vacc_ref[2 * _SL:3 * _SL])

    @pl.when(b == _BSC - 1)
    def _():
        total = jnp.float32(0.0)
        vs = jnp.float32(0.0)
        for bb in range(_B):
            if bb < _BSC:
                sb = acc_ref[bb, 0]
                sf = acc_ref[bb, 1]
                cnt = acc_ref[bb, 2]
            else:
                sb = sums23_ref[bb - _BSC, 0]
                sf = sums23_ref[bb - _BSC, 1]
                cnt = sums23_ref[bb - _BSC, 2]
            denom = jnp.maximum(cnt, 1.0)
            comb = 0.5 * (sb + sf) / denom
            has_valid = (cnt > 0.0).astype(jnp.float32)
            total = total + comb * has_valid
            vs = vs + has_valid
        out_ref[0, 0] = jnp.where(vs > 0.0, total / jnp.maximum(vs, 1.0), total)


def _lossB(x2, gt01, hm2, sums23):
    return pl.pallas_call(
        _lossB_body,
        grid=(_BSC,),
        in_specs=[
            pl.BlockSpec((_Y, _X), lambda b: (b, 0)),
            pl.BlockSpec((_Y, _X), lambda b: (b, 0)),
            pl.BlockSpec((_Y, _X), lambda b: (b, 0)),
            pl.BlockSpec(memory_space=pltpu.SMEM),
        ],
        out_specs=pl.BlockSpec(memory_space=pltpu.SMEM),
        out_shape=jax.ShapeDtypeStruct((1, 1), jnp.float32),
        scratch_shapes=[
            pltpu.SMEM((_BSC, 3), jnp.float32),
            pltpu.VMEM((3 * _SL, _X), jnp.float32),
        ],
    )(x2, gt01, hm2, sums23)


def kernel(attention_logits, gt_bboxes_3d, height_maps):
    params_t = _box_params_t(gt_bboxes_3d)
    x2 = attention_logits.reshape(_B * _Y, _X)
    hm2 = height_maps.reshape(_B * _Y, _X)
    gt01 = _rasterize(params_t).reshape(_BSC * _Y, _X)
    sums23 = _fused23(params_t, x2, hm2)
    return _lossB(x2, gt01, hm2, sums23)[0, 0]
